# baseline (speedup 1.0000x reference)
import jax
import jax.numpy as jnp
from jax import lax
from jax.experimental import pallas as pl
from jax.experimental.pallas import tpu as pltpu
from jax.experimental.pallas import tpu_sc as plsc

VOCAB = 1000000
DIM = 128
BATCH = 4096
SEQ = 200

NC = 2
NS = 16
L = 16
NW = NC * NS

TOTAL = BATCH * SEQ
TOK_PER_W = TOTAL // NW
CH = 128
NCH = TOK_PER_W // CH
VREGS_PER_CH = CH // L
NBUF = 4
LA = 2


def _sc_kernel(tok_hbm, table_hbm, out_hbm, tok_v, idx_v, bufs, gsems, wsems):
    wid = lax.axis_index("c") * NS + lax.axis_index("s")
    base = wid * TOK_PER_W

    pltpu.sync_copy(tok_hbm.at[pl.ds(base, TOK_PER_W)],
                    tok_v.at[pl.ds(8, TOK_PER_W)])

    lanes = lax.iota(jnp.int32, L)

    def hash_chunk(j, b):
        row = idx_v.at[b]
        for l in range(VREGS_PER_CH):
            off = j * CH + l * L
            cur = tok_v[pl.ds(8 + off, L)]
            prev = tok_v[pl.ds(7 + off, L)]
            start_lane = (SEQ - off % SEQ) % SEQ
            prev = jnp.where(lanes == start_lane, 0, prev)
            row[pl.ds(l * L, L)] = (prev * 31 + cur) % VOCAB

    def start_gather(j, b):
        pltpu.async_copy(table_hbm.at[idx_v.at[b]], bufs.at[b], gsems.at[b])

    def wait_gather(b):
        pltpu.make_async_copy(table_hbm.at[idx_v.at[b]], bufs.at[b],
                              gsems.at[b]).wait()

    def start_write(j, b):
        pltpu.async_copy(bufs.at[b], out_hbm.at[pl.ds(base + j * CH, CH)],
                         wsems.at[b])

    def wait_write(j, b):
        pltpu.make_async_copy(bufs.at[b],
                              out_hbm.at[pl.ds(base + j * CH, CH)],
                              wsems.at[b]).wait()

    def visit(j, b, wait_w=True, fire=True):
        if fire:
            bg = (b + LA) % NBUF
            if wait_w:
                wait_write(j + LA - NBUF, bg)
            hash_chunk(j + LA, bg)
            start_gather(j + LA, bg)
        wait_gather(b)
        start_write(j, b)

    for j in range(LA):
        hash_chunk(j, j % NBUF)
        start_gather(j, j % NBUF)
    for j in range(NBUF - LA):
        visit(j, j % NBUF, wait_w=False)
    for j in range(NBUF - LA, NBUF):
        visit(j, j % NBUF)

    first = NBUF
    last_firing = NCH - 1 - LA
    n_groups = (last_firing - first + 1) // NBUF

    def loop_body(j0, _):
        for b in range(NBUF):
            visit(first + j0 * NBUF + b, b)
        return 0

    lax.fori_loop(0, n_groups, loop_body, 0)

    for j in range(first + n_groups * NBUF, last_firing + 1):
        visit(j, j % NBUF)
    for j in range(last_firing + 1, NCH):
        visit(j, j % NBUF, fire=False)

    for j in range(NCH - NBUF, NCH):
        wait_write(j, j % NBUF)


@jax.jit
def kernel(token_ids, embed_weight):
    tok_flat = token_ids.reshape(TOTAL)
    mesh = plsc.VectorSubcoreMesh(core_axis_name="c", subcore_axis_name="s",
                                  num_cores=NC, num_subcores=NS)
    run = pl.kernel(
        _sc_kernel,
        out_type=jax.ShapeDtypeStruct((TOTAL, DIM), jnp.float32),
        mesh=mesh,
        scratch_types=[
            pltpu.VMEM((TOK_PER_W + 8,), jnp.int32),
            pltpu.VMEM((NBUF, CH), jnp.int32),
            pltpu.VMEM((NBUF, CH, DIM), jnp.float32),
            pltpu.SemaphoreType.DMA((NBUF,)),
            pltpu.SemaphoreType.DMA((NBUF,)),
        ],
    )
    out = run(tok_flat, embed_weight)
    return out.reshape(BATCH, SEQ, DIM)
